# baseline (speedup 1.0000x reference)
import functools

import jax
import jax.numpy as jnp
from jax import lax
from jax.experimental import pallas as pl
from jax.experimental.pallas import tpu as pltpu
from jax.experimental.pallas import tpu_sc as plsc

_LANES = 128
_PACK = 8
_GRP = _LANES // _PACK
_NEG = -1e30


def _proj_body(tt_ref, wp_ref, b2_ref, out_ref):
    x = lax.dot_general(
        tt_ref[...], wp_ref[...],
        dimension_numbers=(((0,), (0,)), ((), ())),
        preferred_element_type=jnp.float32,
    )
    x3 = x.reshape(x.shape[0] // _PACK, _PACK, _GRP)
    for r in range(_PACK):
        sl = slice(r * _GRP, (r + 1) * _GRP)
        out_ref[:, sl] = x3[:, r, :] + b2_ref[:, sl]


def _make_sc_gather(prows, batch):
    info = plsc.get_sparse_core_info()
    nc, ns = info.num_cores, info.num_subcores
    nw = nc * ns
    assert batch % (8 * nw) == 0
    b_per_w = batch // nw
    mesh = plsc.VectorSubcoreMesh(core_axis_name="c", subcore_axis_name="s")

    @functools.partial(
        pl.kernel,
        mesh=mesh,
        out_type=jax.ShapeDtypeStruct((batch, _LANES), jnp.float32),
        scratch_types=[
            pltpu.VMEM((b_per_w,), jnp.int32),
            pltpu.VMEM((b_per_w, _LANES), jnp.float32),
            pltpu.SemaphoreType.DMA,
        ],
    )
    def gather_rows(idx_hbm, p_hbm, out_hbm, idx_v, rows_v, sem):
        wid = lax.axis_index("s") * nc + lax.axis_index("c")
        base = wid * b_per_w
        pltpu.sync_copy(idx_hbm.at[pl.ds(base, b_per_w)], idx_v)
        pltpu.async_copy(p_hbm.at[idx_v], rows_v, sem).wait()
        pltpu.sync_copy(rows_v, out_hbm.at[pl.ds(base, b_per_w)])

    return gather_rows


def _loss_body(gath_ref, g_ref, labels_ref, logits_ref, loss_ref):
    gath = gath_ref[...]
    batch = gath.shape[0]
    base = g_ref[...] * _GRP
    lane = lax.broadcasted_iota(jnp.int32, (batch, _LANES), 1)
    ingrp = (lane >= base) & (lane < base + 2)
    vals = jnp.where(ingrp, gath, _NEG)
    m = jnp.max(vals, axis=1, keepdims=True)
    lse = m[:, 0] + jnp.log(jnp.sum(jnp.exp(vals - m), axis=1))
    l0 = jnp.sum(jnp.where(lane == base, gath, 0.0), axis=1)
    l1 = jnp.sum(jnp.where(lane == base + 1, gath, 0.0), axis=1)
    logits_ref[...] = jnp.concatenate([l0[:, None], l1[:, None]], axis=1)
    picked = jnp.where(labels_ref[...][:, 0] == 0, l0, l1)
    loss_ref[0, 0] = jnp.mean(lse - picked)


def kernel(input_ids, labels, emb_table, W, b):
    batch = input_ids.shape[0]
    vocab, dim = emb_table.shape
    num_labels = W.shape[0]
    prows = vocab // _PACK
    blk = 2048
    grid = -(-vocab // blk)

    idx = input_ids[:, 0]
    tt = emb_table.T

    wp = jnp.zeros((dim, _GRP), jnp.float32).at[:, :num_labels].set(W.T)
    b16 = jnp.full((_GRP,), _NEG, jnp.float32).at[:num_labels].set(b)
    b2 = jnp.tile(b16, _PACK)[None, :]

    packed = pl.pallas_call(
        _proj_body,
        grid=(grid,),
        in_specs=[
            pl.BlockSpec((dim, blk), lambda i: (0, i)),
            pl.BlockSpec((dim, _GRP), lambda i: (0, 0)),
            pl.BlockSpec((1, _LANES), lambda i: (0, 0)),
        ],
        out_specs=pl.BlockSpec((blk // _PACK, _LANES), lambda i: (i, 0)),
        out_shape=jax.ShapeDtypeStruct((prows, _LANES), jnp.float32),
    )(tt, wp, b2)

    gath = _make_sc_gather(prows, batch)(idx >> 3, packed)

    logits, loss = pl.pallas_call(
        _loss_body,
        out_shape=(
            jax.ShapeDtypeStruct((batch, num_labels), jnp.float32),
            jax.ShapeDtypeStruct((1, 1), jnp.float32),
        ),
        in_specs=[pl.BlockSpec(memory_space=pltpu.VMEM)] * 3,
        out_specs=(
            pl.BlockSpec(memory_space=pltpu.VMEM),
            pl.BlockSpec(memory_space=pltpu.SMEM),
        ),
    )(gath, (idx & 7)[:, None], labels[:, None].astype(jnp.int32))

    return loss[0, 0], logits


# speedup vs baseline: 1.7203x; 1.7203x over previous
import functools

import jax
import jax.numpy as jnp
from jax import lax
from jax.experimental import pallas as pl
from jax.experimental.pallas import tpu as pltpu
from jax.experimental.pallas import tpu_sc as plsc

_LANES = 128
_NEG = -1e30


def _proj_body(tt_ref, wp_ref, b2_ref, out_ref):
    x = lax.dot_general(
        tt_ref[...], wp_ref[...],
        dimension_numbers=(((0,), (0,)), ((), ())),
        preferred_element_type=jnp.float32,
    )
    out_ref[...] = x + b2_ref[...]


def _make_sc_gather(prows, batch):
    info = plsc.get_sparse_core_info()
    nc, ns = info.num_cores, info.num_subcores
    nw = nc * ns
    assert batch % (8 * nw) == 0
    b_per_w = batch // nw
    mesh = plsc.VectorSubcoreMesh(core_axis_name="c", subcore_axis_name="s")

    @functools.partial(
        pl.kernel,
        mesh=mesh,
        out_type=jax.ShapeDtypeStruct((batch, _LANES), jnp.float32),
        scratch_types=[
            pltpu.VMEM((b_per_w,), jnp.int32),
            pltpu.VMEM((b_per_w, _LANES), jnp.float32),
            pltpu.SemaphoreType.DMA,
        ],
    )
    def gather_rows(idx_hbm, p_hbm, out_hbm, idx_v, rows_v, sem):
        wid = lax.axis_index("s") * nc + lax.axis_index("c")
        base = wid * b_per_w
        pltpu.sync_copy(idx_hbm.at[pl.ds(base, b_per_w)], idx_v)
        pltpu.async_copy(p_hbm.at[idx_v], rows_v, sem).wait()
        pltpu.sync_copy(rows_v, out_hbm.at[pl.ds(base, b_per_w)])

    return gather_rows


def _loss_body(gath_ref, labels_ref, logits_ref, loss_ref):
    gath = gath_ref[...]
    batch = gath.shape[0]
    lane = lax.broadcasted_iota(jnp.int32, (batch, _LANES), 1)
    m = jnp.max(gath, axis=1, keepdims=True)
    lse = m[:, 0] + jnp.log(jnp.sum(jnp.exp(gath - m), axis=1))
    l0 = jnp.sum(jnp.where(lane == 0, gath, 0.0), axis=1)
    l1 = jnp.sum(jnp.where(lane == 1, gath, 0.0), axis=1)
    logits_ref[...] = jnp.concatenate([l0[:, None], l1[:, None]], axis=1)
    picked = jnp.where(labels_ref[...][:, 0] == 0, l0, l1)
    loss_ref[0, 0] = jnp.mean(lse - picked)


def kernel(input_ids, labels, emb_table, W, b):
    batch = input_ids.shape[0]
    vocab, dim = emb_table.shape
    num_labels = W.shape[0]
    blk = 8192
    grid = -(-vocab // blk)

    idx = input_ids[:, 0]
    tt = emb_table.T

    wp = jnp.zeros((dim, _LANES), jnp.float32).at[:, :num_labels].set(W.T)
    b2 = jnp.full((1, _LANES), _NEG, jnp.float32).at[0, :num_labels].set(b)

    packed = pl.pallas_call(
        _proj_body,
        grid=(grid,),
        in_specs=[
            pl.BlockSpec((dim, blk), lambda i: (0, i)),
            pl.BlockSpec((dim, _LANES), lambda i: (0, 0)),
            pl.BlockSpec((1, _LANES), lambda i: (0, 0)),
        ],
        out_specs=pl.BlockSpec((blk, _LANES), lambda i: (i, 0)),
        out_shape=jax.ShapeDtypeStruct((vocab, _LANES), jnp.float32),
    )(tt, wp, b2)

    gath = _make_sc_gather(vocab, batch)(idx, packed)

    logits, loss = pl.pallas_call(
        _loss_body,
        out_shape=(
            jax.ShapeDtypeStruct((batch, num_labels), jnp.float32),
            jax.ShapeDtypeStruct((1, 1), jnp.float32),
        ),
        in_specs=[pl.BlockSpec(memory_space=pltpu.VMEM)] * 2,
        out_specs=(
            pl.BlockSpec(memory_space=pltpu.VMEM),
            pl.BlockSpec(memory_space=pltpu.SMEM),
        ),
    )(gath, labels[:, None].astype(jnp.int32))

    return loss[0, 0], logits


# speedup vs baseline: 1.7570x; 1.0213x over previous
import functools

import jax
import jax.numpy as jnp
from jax import lax
from jax.experimental import pallas as pl
from jax.experimental.pallas import tpu as pltpu
from jax.experimental.pallas import tpu_sc as plsc

_LANES = 128
_NEG = -1e30


def _proj_body(tt_ref, wp_ref, b2_ref, out_ref):
    x = lax.dot_general(
        tt_ref[...], wp_ref[...],
        dimension_numbers=(((0,), (0,)), ((), ())),
        preferred_element_type=jnp.float32,
    )
    out_ref[...] = x + b2_ref[...]


def _make_sc_gather(prows, batch):
    info = plsc.get_sparse_core_info()
    nc, ns = info.num_cores, info.num_subcores
    nw = nc * ns
    assert batch % (8 * nw) == 0
    b_per_w = batch // nw
    mesh = plsc.VectorSubcoreMesh(core_axis_name="c", subcore_axis_name="s")

    @functools.partial(
        pl.kernel,
        mesh=mesh,
        out_type=jax.ShapeDtypeStruct((batch, _LANES), jnp.float32),
        scratch_types=[
            pltpu.VMEM((b_per_w,), jnp.int32),
            pltpu.VMEM((b_per_w, _LANES), jnp.float32),
            pltpu.SemaphoreType.DMA,
        ],
    )
    def gather_rows(idx_hbm, p_hbm, out_hbm, idx_v, rows_v, sem):
        wid = lax.axis_index("s") * nc + lax.axis_index("c")
        base = wid * b_per_w
        pltpu.sync_copy(idx_hbm.at[pl.ds(base, b_per_w)], idx_v)
        pltpu.async_copy(p_hbm.at[idx_v], rows_v, sem).wait()
        pltpu.sync_copy(rows_v, out_hbm.at[pl.ds(base, b_per_w)])

    return gather_rows


def _loss_body(gath_ref, labels_ref, logits_ref, loss_ref):
    gath = gath_ref[...]
    batch = gath.shape[0]
    lane = lax.broadcasted_iota(jnp.int32, (batch, _LANES), 1)
    m = jnp.max(gath, axis=1, keepdims=True)
    lse = m[:, 0] + jnp.log(jnp.sum(jnp.exp(gath - m), axis=1))
    l0 = jnp.sum(jnp.where(lane == 0, gath, 0.0), axis=1)
    l1 = jnp.sum(jnp.where(lane == 1, gath, 0.0), axis=1)
    logits_ref[...] = jnp.concatenate([l0[:, None], l1[:, None]], axis=1)
    picked = jnp.where(labels_ref[...][:, 0] == 0, l0, l1)
    loss_ref[0, 0] = jnp.mean(lse - picked)


def kernel(input_ids, labels, emb_table, W, b):
    batch = input_ids.shape[0]
    vocab, dim = emb_table.shape
    num_labels = W.shape[0]
    blk = 16384
    grid = -(-vocab // blk)

    idx = input_ids[:, 0]
    tt = emb_table.T

    wp = jnp.zeros((dim, _LANES), jnp.float32).at[:, :num_labels].set(W.T)
    b2 = jnp.full((1, _LANES), _NEG, jnp.float32).at[0, :num_labels].set(b)

    packed = pl.pallas_call(
        _proj_body,
        grid=(grid,),
        in_specs=[
            pl.BlockSpec((dim, blk), lambda i: (0, i)),
            pl.BlockSpec((dim, _LANES), lambda i: (0, 0)),
            pl.BlockSpec((1, _LANES), lambda i: (0, 0)),
        ],
        out_specs=pl.BlockSpec((blk, _LANES), lambda i: (i, 0)),
        out_shape=jax.ShapeDtypeStruct((vocab, _LANES), jnp.float32),
    )(tt, wp, b2)

    gath = _make_sc_gather(vocab, batch)(idx, packed)

    logits, loss = pl.pallas_call(
        _loss_body,
        out_shape=(
            jax.ShapeDtypeStruct((batch, num_labels), jnp.float32),
            jax.ShapeDtypeStruct((1, 1), jnp.float32),
        ),
        in_specs=[pl.BlockSpec(memory_space=pltpu.VMEM)] * 2,
        out_specs=(
            pl.BlockSpec(memory_space=pltpu.VMEM),
            pl.BlockSpec(memory_space=pltpu.SMEM),
        ),
    )(gath, labels[:, None].astype(jnp.int32))

    return loss[0, 0], logits


# speedup vs baseline: 1.7886x; 1.0180x over previous
import functools

import jax
import jax.numpy as jnp
from jax import lax
from jax.experimental import pallas as pl
from jax.experimental.pallas import tpu as pltpu
from jax.experimental.pallas import tpu_sc as plsc

_LANES = 128
_NEG = -1e30


def _proj_body(tt_ref, wp_ref, b2_ref, out_ref):
    x = lax.dot_general(
        tt_ref[...], wp_ref[...],
        dimension_numbers=(((0,), (0,)), ((), ())),
        preferred_element_type=jnp.float32,
    )
    out_ref[...] = x + b2_ref[...]


def _make_sc_gather(prows, batch):
    info = plsc.get_sparse_core_info()
    nc, ns = info.num_cores, info.num_subcores
    nw = nc * ns
    assert batch % (8 * nw) == 0
    b_per_w = batch // nw
    mesh = plsc.VectorSubcoreMesh(core_axis_name="c", subcore_axis_name="s")

    @functools.partial(
        pl.kernel,
        mesh=mesh,
        out_type=jax.ShapeDtypeStruct((batch, _LANES), jnp.float32),
        scratch_types=[
            pltpu.VMEM((b_per_w,), jnp.int32),
            pltpu.VMEM((b_per_w, _LANES), jnp.float32),
            pltpu.SemaphoreType.DMA,
        ],
    )
    def gather_rows(idx_hbm, p_hbm, out_hbm, idx_v, rows_v, sem):
        wid = lax.axis_index("s") * nc + lax.axis_index("c")
        base = wid * b_per_w
        pltpu.sync_copy(idx_hbm.at[pl.ds(base, b_per_w)], idx_v)
        pltpu.async_copy(p_hbm.at[idx_v], rows_v, sem).wait()
        pltpu.sync_copy(rows_v, out_hbm.at[pl.ds(base, b_per_w)])

    return gather_rows


def _loss_body(gath_ref, labels_ref, logits_ref, loss_ref):
    gath = gath_ref[...]
    batch = gath.shape[0]
    lane = lax.broadcasted_iota(jnp.int32, (batch, _LANES), 1)
    m = jnp.max(gath, axis=1, keepdims=True)
    lse = m[:, 0] + jnp.log(jnp.sum(jnp.exp(gath - m), axis=1))
    l0 = jnp.sum(jnp.where(lane == 0, gath, 0.0), axis=1)
    l1 = jnp.sum(jnp.where(lane == 1, gath, 0.0), axis=1)
    logits_ref[...] = jnp.concatenate([l0[:, None], l1[:, None]], axis=1)
    picked = jnp.where(labels_ref[...][:, 0] == 0, l0, l1)
    loss_ref[0, 0] = jnp.mean(lse - picked)


def kernel(input_ids, labels, emb_table, W, b):
    batch = input_ids.shape[0]
    vocab, dim = emb_table.shape
    num_labels = W.shape[0]
    blk = 32768
    grid = -(-vocab // blk)

    idx = input_ids[:, 0]
    tt = emb_table.T

    wp = jnp.zeros((dim, _LANES), jnp.float32).at[:, :num_labels].set(W.T)
    b2 = jnp.full((1, _LANES), _NEG, jnp.float32).at[0, :num_labels].set(b)

    packed = pl.pallas_call(
        _proj_body,
        grid=(grid,),
        in_specs=[
            pl.BlockSpec((dim, blk), lambda i: (0, i)),
            pl.BlockSpec((dim, _LANES), lambda i: (0, 0)),
            pl.BlockSpec((1, _LANES), lambda i: (0, 0)),
        ],
        out_specs=pl.BlockSpec((blk, _LANES), lambda i: (i, 0)),
        out_shape=jax.ShapeDtypeStruct((vocab, _LANES), jnp.float32),
        compiler_params=pltpu.CompilerParams(vmem_limit_bytes=100 << 20),
    )(tt, wp, b2)

    gath = _make_sc_gather(vocab, batch)(idx, packed)

    logits, loss = pl.pallas_call(
        _loss_body,
        out_shape=(
            jax.ShapeDtypeStruct((batch, num_labels), jnp.float32),
            jax.ShapeDtypeStruct((1, 1), jnp.float32),
        ),
        in_specs=[pl.BlockSpec(memory_space=pltpu.VMEM)] * 2,
        out_specs=(
            pl.BlockSpec(memory_space=pltpu.VMEM),
            pl.BlockSpec(memory_space=pltpu.SMEM),
        ),
    )(gath, labels[:, None].astype(jnp.int32))

    return loss[0, 0], logits


# speedup vs baseline: 1.8125x; 1.0134x over previous
import functools

import jax
import jax.numpy as jnp
from jax import lax
from jax.experimental import pallas as pl
from jax.experimental.pallas import tpu as pltpu
from jax.experimental.pallas import tpu_sc as plsc

_LANES = 128
_NEG = -1e30


def _proj_body(tt_ref, wp_ref, b2_ref, out_ref):
    x = lax.dot_general(
        tt_ref[...], wp_ref[...],
        dimension_numbers=(((0,), (0,)), ((), ())),
        preferred_element_type=jnp.float32,
    )
    out_ref[...] = x + b2_ref[...]


def _make_sc_gather(prows, batch):
    info = plsc.get_sparse_core_info()
    nc, ns = info.num_cores, info.num_subcores
    nw = nc * ns
    assert batch % (8 * nw) == 0
    b_per_w = batch // nw
    mesh = plsc.VectorSubcoreMesh(core_axis_name="c", subcore_axis_name="s")

    @functools.partial(
        pl.kernel,
        mesh=mesh,
        out_type=jax.ShapeDtypeStruct((batch, _LANES), jnp.float32),
        scratch_types=[
            pltpu.VMEM((b_per_w,), jnp.int32),
            pltpu.VMEM((b_per_w, _LANES), jnp.float32),
            pltpu.SemaphoreType.DMA,
        ],
    )
    def gather_rows(idx_hbm, p_hbm, out_hbm, idx_v, rows_v, sem):
        wid = lax.axis_index("s") * nc + lax.axis_index("c")
        base = wid * b_per_w
        pltpu.sync_copy(idx_hbm.at[pl.ds(base, b_per_w)], idx_v)
        pltpu.async_copy(p_hbm.at[idx_v], rows_v, sem).wait()
        pltpu.sync_copy(rows_v, out_hbm.at[pl.ds(base, b_per_w)])

    return gather_rows


def _loss_body(gath_ref, labels_ref, logits_ref, loss_ref):
    gath = gath_ref[...]
    batch = gath.shape[0]
    lane = lax.broadcasted_iota(jnp.int32, (batch, _LANES), 1)
    m = jnp.max(gath, axis=1, keepdims=True)
    lse = m[:, 0] + jnp.log(jnp.sum(jnp.exp(gath - m), axis=1))
    logits_ref[...] = gath[:, :2]
    picked = jnp.sum(jnp.where(lane == labels_ref[...], gath, 0.0), axis=1)
    loss_ref[0, 0] = jnp.mean(lse - picked)


def kernel(input_ids, labels, emb_table, W, b):
    batch = input_ids.shape[0]
    vocab, dim = emb_table.shape
    num_labels = W.shape[0]
    blk = 32768
    grid = -(-vocab // blk)

    idx = input_ids[:, 0]
    tt = emb_table.T

    wp = jnp.zeros((dim, _LANES), jnp.float32).at[:, :num_labels].set(W.T)
    b2 = jnp.full((1, _LANES), _NEG, jnp.float32).at[0, :num_labels].set(b)

    packed = pl.pallas_call(
        _proj_body,
        grid=(grid,),
        in_specs=[
            pl.BlockSpec((dim, blk), lambda i: (0, i)),
            pl.BlockSpec((dim, _LANES), lambda i: (0, 0)),
            pl.BlockSpec((1, _LANES), lambda i: (0, 0)),
        ],
        out_specs=pl.BlockSpec((blk, _LANES), lambda i: (i, 0)),
        out_shape=jax.ShapeDtypeStruct((vocab, _LANES), jnp.float32),
        compiler_params=pltpu.CompilerParams(vmem_limit_bytes=100 << 20),
    )(tt, wp, b2)

    gath = _make_sc_gather(vocab, batch)(idx, packed)

    logits, loss = pl.pallas_call(
        _loss_body,
        out_shape=(
            jax.ShapeDtypeStruct((batch, num_labels), jnp.float32),
            jax.ShapeDtypeStruct((1, 1), jnp.float32),
        ),
        in_specs=[pl.BlockSpec(memory_space=pltpu.VMEM)] * 2,
        out_specs=(
            pl.BlockSpec(memory_space=pltpu.VMEM),
            pl.BlockSpec(memory_space=pltpu.SMEM),
        ),
    )(gath, labels[:, None].astype(jnp.int32))

    return loss[0, 0], logits


# speedup vs baseline: 3.1086x; 1.7151x over previous
import functools

import jax
import jax.numpy as jnp
from jax import lax
from jax.experimental import pallas as pl
from jax.experimental.pallas import tpu as pltpu
from jax.experimental.pallas import tpu_sc as plsc

_LANES = 128


def _proj_body(tt_ref, w8_ref, b8_ref, out_ref):
    y = jnp.dot(w8_ref[...], tt_ref[...],
                preferred_element_type=jnp.float32) + b8_ref[...]
    yb = y.astype(jnp.bfloat16)
    u0 = lax.bitcast_convert_type(yb[0:1, :], jnp.uint16).astype(jnp.uint32)
    u1 = lax.bitcast_convert_type(yb[1:2, :], jnp.uint16).astype(jnp.uint32)
    w = ((u0 << 16) | u1)[0, :]
    out_ref[...] = lax.bitcast_convert_type(w, jnp.int32)


def _make_sc_gather(pwords, batch):
    info = plsc.get_sparse_core_info()
    nc, ns = info.num_cores, info.num_subcores
    nw = nc * ns
    assert batch % (8 * nw) == 0
    b_per_w = batch // nw
    mesh = plsc.VectorSubcoreMesh(core_axis_name="c", subcore_axis_name="s")

    @functools.partial(
        pl.kernel,
        mesh=mesh,
        out_type=jax.ShapeDtypeStruct((batch,), jnp.int32),
        scratch_types=[
            pltpu.VMEM((b_per_w,), jnp.int32),
            pltpu.VMEM((b_per_w,), jnp.int32),
            pltpu.SemaphoreType.DMA,
        ],
    )
    def gather_rows(idx_hbm, p_hbm, out_hbm, idx_v, words_v, sem):
        wid = lax.axis_index("s") * nc + lax.axis_index("c")
        base = wid * b_per_w
        pltpu.sync_copy(idx_hbm.at[pl.ds(base, b_per_w)], idx_v)
        pltpu.async_copy(p_hbm.at[idx_v], words_v, sem).wait()
        pltpu.sync_copy(words_v, out_hbm.at[pl.ds(base, b_per_w)])

    return gather_rows


def _loss_body(gath_ref, labels_ref, l0_ref, l1_ref, loss_ref):
    u = lax.bitcast_convert_type(gath_ref[...], jnp.uint32)
    l0 = lax.bitcast_convert_type(
        (u >> 16).astype(jnp.uint16), jnp.bfloat16).astype(jnp.float32)
    l1 = lax.bitcast_convert_type(
        (u & 0xFFFF).astype(jnp.uint16), jnp.bfloat16).astype(jnp.float32)
    m = jnp.maximum(l0, l1)
    lse = m + jnp.log(jnp.exp(l0 - m) + jnp.exp(l1 - m))
    picked = jnp.where(labels_ref[...] == 0, l0, l1)
    l0_ref[...] = l0
    l1_ref[...] = l1
    loss_ref[0, 0] = jnp.mean(lse - picked)


def kernel(input_ids, labels, emb_table, W, b):
    batch = input_ids.shape[0]
    vocab, dim = emb_table.shape
    num_labels = W.shape[0]
    blk = 32768
    grid = -(-vocab // blk)
    rows = batch // _LANES

    idx = input_ids[:, 0]
    tt = emb_table.T

    w8 = jnp.zeros((8, dim), jnp.float32).at[:num_labels, :].set(W)
    b8 = jnp.zeros((8, 1), jnp.float32).at[:num_labels, 0].set(b)

    packed = pl.pallas_call(
        _proj_body,
        grid=(grid,),
        in_specs=[
            pl.BlockSpec((dim, blk), lambda i: (0, i)),
            pl.BlockSpec((8, dim), lambda i: (0, 0)),
            pl.BlockSpec((8, 1), lambda i: (0, 0)),
        ],
        out_specs=pl.BlockSpec((blk,), lambda i: (i,)),
        out_shape=jax.ShapeDtypeStruct((vocab,), jnp.int32),
        compiler_params=pltpu.CompilerParams(vmem_limit_bytes=100 << 20),
    )(tt, w8, b8)

    gath = _make_sc_gather(vocab, batch)(idx, packed)

    l0, l1, loss = pl.pallas_call(
        _loss_body,
        out_shape=(
            jax.ShapeDtypeStruct((rows, _LANES), jnp.float32),
            jax.ShapeDtypeStruct((rows, _LANES), jnp.float32),
            jax.ShapeDtypeStruct((1, 1), jnp.float32),
        ),
        in_specs=[pl.BlockSpec(memory_space=pltpu.VMEM)] * 2,
        out_specs=(
            pl.BlockSpec(memory_space=pltpu.VMEM),
            pl.BlockSpec(memory_space=pltpu.VMEM),
            pl.BlockSpec(memory_space=pltpu.SMEM),
        ),
    )(gath.reshape(rows, _LANES), labels.reshape(rows, _LANES))

    logits = jnp.stack([l0.reshape(batch), l1.reshape(batch)], axis=1)
    return loss[0, 0], logits


# speedup vs baseline: 3.3562x; 1.0797x over previous
import functools

import jax
import jax.numpy as jnp
from jax import lax
from jax.experimental import pallas as pl
from jax.experimental.pallas import tpu as pltpu
from jax.experimental.pallas import tpu_sc as plsc

_LANES = 128


def _proj_body(tt_ref, w_ref, b_ref, out_ref):
    y = jnp.dot(w_ref[...], tt_ref[...],
                preferred_element_type=jnp.float32) + b_ref[...]
    yb = y.astype(jnp.bfloat16)
    u0 = lax.bitcast_convert_type(yb[0:1, :], jnp.uint16).astype(jnp.uint32)
    u1 = lax.bitcast_convert_type(yb[1:2, :], jnp.uint16).astype(jnp.uint32)
    w = ((u0 << 16) | u1)[0, :]
    out_ref[...] = lax.bitcast_convert_type(w, jnp.int32)


def _make_sc_gather(pwords, batch):
    info = plsc.get_sparse_core_info()
    nc, ns = info.num_cores, info.num_subcores
    nw = nc * ns
    assert batch % (8 * nw) == 0
    b_per_w = batch // nw
    mesh = plsc.VectorSubcoreMesh(core_axis_name="c", subcore_axis_name="s")

    @functools.partial(
        pl.kernel,
        mesh=mesh,
        out_type=jax.ShapeDtypeStruct((batch,), jnp.int32),
        scratch_types=[
            pltpu.VMEM((b_per_w,), jnp.int32),
            pltpu.VMEM((b_per_w,), jnp.int32),
            pltpu.SemaphoreType.DMA,
        ],
    )
    def gather_rows(tids_hbm, p_hbm, out_hbm, idx_v, words_v, sem):
        wid = lax.axis_index("s") * nc + lax.axis_index("c")
        base = wid * b_per_w
        pltpu.sync_copy(tids_hbm.at[0, pl.ds(base, b_per_w)], idx_v)
        pltpu.async_copy(p_hbm.at[idx_v], words_v, sem).wait()
        pltpu.sync_copy(words_v, out_hbm.at[pl.ds(base, b_per_w)])

    return gather_rows


def _loss_body(gath_ref, labels_ref, l0_ref, l1_ref, loss_ref):
    u = lax.bitcast_convert_type(gath_ref[...], jnp.uint32)
    l0 = lax.bitcast_convert_type(
        (u >> 16).astype(jnp.uint16), jnp.bfloat16).astype(jnp.float32)
    l1 = lax.bitcast_convert_type(
        (u & 0xFFFF).astype(jnp.uint16), jnp.bfloat16).astype(jnp.float32)
    m = jnp.maximum(l0, l1)
    lse = m + jnp.log(jnp.exp(l0 - m) + jnp.exp(l1 - m))
    picked = jnp.where(labels_ref[...] == 0, l0, l1)
    l0_ref[...] = l0
    l1_ref[...] = l1
    loss_ref[0, 0] = jnp.mean(lse - picked)


def kernel(input_ids, labels, emb_table, W, b):
    batch = input_ids.shape[0]
    vocab, dim = emb_table.shape
    num_labels = W.shape[0]
    blk = 32768
    grid = -(-vocab // blk)
    rows = batch // _LANES

    tt = emb_table.T
    tids = input_ids.T

    packed = pl.pallas_call(
        _proj_body,
        grid=(grid,),
        in_specs=[
            pl.BlockSpec((dim, blk), lambda i: (0, i)),
            pl.BlockSpec((num_labels, dim), lambda i: (0, 0)),
            pl.BlockSpec((num_labels, 1), lambda i: (0, 0)),
        ],
        out_specs=pl.BlockSpec((blk,), lambda i: (i,)),
        out_shape=jax.ShapeDtypeStruct((vocab,), jnp.int32),
        compiler_params=pltpu.CompilerParams(vmem_limit_bytes=100 << 20),
    )(tt, W, b[:, None])

    gath = _make_sc_gather(vocab, batch)(tids, packed)

    l0, l1, loss = pl.pallas_call(
        _loss_body,
        out_shape=(
            jax.ShapeDtypeStruct((rows, _LANES), jnp.float32),
            jax.ShapeDtypeStruct((rows, _LANES), jnp.float32),
            jax.ShapeDtypeStruct((1, 1), jnp.float32),
        ),
        in_specs=[pl.BlockSpec(memory_space=pltpu.VMEM)] * 2,
        out_specs=(
            pl.BlockSpec(memory_space=pltpu.VMEM),
            pl.BlockSpec(memory_space=pltpu.VMEM),
            pl.BlockSpec(memory_space=pltpu.SMEM),
        ),
    )(gath.reshape(rows, _LANES), labels.reshape(rows, _LANES))

    logits = jnp.stack([l0.reshape(batch), l1.reshape(batch)], axis=1)
    return loss[0, 0], logits
